# Initial kernel scaffold; baseline (speedup 1.0000x reference)
import jax
import jax.numpy as jnp
from jax.experimental import pallas as pl


def kernel(x, t, W, b):
    raise NotImplementedError("write your pallas kernel here")


# speedup vs baseline: 4.2284x; 4.2284x over previous
import jax
import jax.numpy as jnp
from jax.experimental import pallas as pl

_N = 8192
_D = 768
_MT = 16
_MY = 32
_BLK = 1024


def _body(x_ref, t_ref, w_ref, b_ref, o_ref):
    x = x_ref[...]
    w = w_ref[...]
    logits = jax.lax.dot_general(
        x, w, (((1,), (1,)), ((), ())), preferred_element_type=jnp.float32
    )
    logits = logits + b_ref[...]
    tt = t_ref[...]
    sel = jnp.zeros((x.shape[0], _MY), jnp.float32)
    for e in range(_MT):
        sel = sel + jnp.where(tt == e, logits[:, e * _MY:(e + 1) * _MY], 0.0)
    m = jnp.max(sel, axis=1, keepdims=True)
    p = jnp.exp(sel - m)
    o_ref[...] = p / jnp.sum(p, axis=1, keepdims=True)


def kernel(x, t, W, b):
    n, d = x.shape
    w2 = W.reshape(_MT * _MY, d)
    b2 = b.reshape(1, _MT * _MY)
    t2 = t.reshape(n, 1)
    grid = (n // _BLK,)
    return pl.pallas_call(
        _body,
        grid=grid,
        in_specs=[
            pl.BlockSpec((_BLK, d), lambda i: (i, 0)),
            pl.BlockSpec((_BLK, 1), lambda i: (i, 0)),
            pl.BlockSpec((_MT * _MY, d), lambda i: (0, 0)),
            pl.BlockSpec((1, _MT * _MY), lambda i: (0, 0)),
        ],
        out_specs=pl.BlockSpec((_BLK, _MY), lambda i: (i, 0)),
        out_shape=jax.ShapeDtypeStruct((n, _MY), x.dtype),
    )(x, t2, w2, b2)
